# initial kernel scaffold (unmeasured)
import jax
import jax.numpy as jnp
from jax import lax
from jax.experimental import pallas as pl
from jax.experimental.pallas import tpu as pltpu

N_DEV = 4
SQ = 2048
D_MODEL = 1024
HQ_LOCAL = 8
DH = 128
HD_LOCAL = HQ_LOCAL * DH
WINDOW = 128
SCALE = 0.08838834764831843
QBLK = 512
KBLK = QBLK + 2 * WINDOW


def _attn_body(x_ref, wq_ref, k_ref, v_ref, wo_ref, out_ref, ctx_ref):
    q = jnp.dot(x_ref[:], wq_ref[:], preferred_element_type=jnp.float32)
    for qb in range(SQ // QBLK):
        qs = qb * QBLK
        ks = min(max(qs - WINDOW, 0), SQ - KBLK)
        rows = lax.broadcasted_iota(jnp.int32, (QBLK, KBLK), 0) + qs
        cols = lax.broadcasted_iota(jnp.int32, (QBLK, KBLK), 1) + ks
        band = jnp.abs(rows - cols) <= WINDOW
        for h in range(HQ_LOCAL):
            qh = q[qs : qs + QBLK, h * DH : (h + 1) * DH]
            kh = k_ref[h, ks : ks + KBLK, :]
            s = lax.dot_general(
                qh, kh, (((1,), (1,)), ((), ())),
                preferred_element_type=jnp.float32,
            ) * SCALE
            s = jnp.where(band, s, -1e9)
            m = jnp.max(s, axis=1, keepdims=True)
            w = jnp.exp(s - m)
            w = w / jnp.sum(w, axis=1, keepdims=True)
            vh = v_ref[h, ks : ks + KBLK, :]
            ctx_ref[qs : qs + QBLK, h * DH : (h + 1) * DH] = jnp.dot(
                w, vh, preferred_element_type=jnp.float32
            )
    out_ref[:] = jnp.dot(ctx_ref[:], wo_ref[:], preferred_element_type=jnp.float32)


def _ar_body(p_ref, out_ref, comm_ref, send_sems, recv_sems):
    my_pos = lax.axis_index("i")
    left = (my_pos - 1) % N_DEV
    right = (my_pos + 1) % N_DEV

    barrier_sem = pltpu.get_barrier_semaphore()
    for nbr in [left, right]:
        pl.semaphore_signal(
            barrier_sem, inc=1,
            device_id=(nbr,), device_id_type=pl.DeviceIdType.MESH,
        )
    pl.semaphore_wait(barrier_sem, 2)

    out_ref[:] = p_ref[:]
    comm_ref[0] = p_ref[:]

    for h in range(N_DEV - 1):
        send_slot = h % 2
        recv_slot = (h + 1) % 2
        rdma = pltpu.make_async_remote_copy(
            src_ref=comm_ref.at[send_slot],
            dst_ref=comm_ref.at[recv_slot],
            send_sem=send_sems.at[send_slot],
            recv_sem=recv_sems.at[recv_slot],
            device_id=(right,),
            device_id_type=pl.DeviceIdType.MESH,
        )
        rdma.start()
        rdma.wait()
        out_ref[:] += comm_ref[recv_slot]


def kernel(x, Wq, K_ext, V_ext, Wo):
    my_pos = lax.axis_index("i")
    x2 = x.reshape(SQ, D_MODEL)
    wq_l = lax.dynamic_slice(Wq, (0, my_pos * HD_LOCAL), (D_MODEL, HD_LOCAL))
    wo_l = lax.dynamic_slice(Wo, (my_pos * HD_LOCAL, 0), (HD_LOCAL, D_MODEL))
    k = K_ext.reshape(SQ, HQ_LOCAL, DH).transpose(1, 0, 2)
    v = V_ext.reshape(SQ, HQ_LOCAL, DH).transpose(1, 0, 2)

    partial = pl.pallas_call(
        _attn_body,
        out_shape=jax.ShapeDtypeStruct((SQ, D_MODEL), jnp.float32),
        in_specs=[pl.BlockSpec(memory_space=pltpu.VMEM)] * 5,
        out_specs=pl.BlockSpec(memory_space=pltpu.VMEM),
        scratch_shapes=[pltpu.VMEM((SQ, HD_LOCAL), jnp.float32)],
    )(x2, wq_l, k, v, wo_l)

    out = pl.pallas_call(
        _ar_body,
        out_shape=jax.ShapeDtypeStruct((SQ, D_MODEL), jnp.float32),
        in_specs=[pl.BlockSpec(memory_space=pltpu.VMEM)],
        out_specs=pl.BlockSpec(memory_space=pltpu.VMEM),
        scratch_shapes=[
            pltpu.VMEM((2, SQ, D_MODEL), jnp.float32),
            pltpu.SemaphoreType.DMA((2,)),
            pltpu.SemaphoreType.DMA((2,)),
        ],
        compiler_params=pltpu.CompilerParams(collective_id=0),
    )(partial)
    return out.reshape(1, SQ, D_MODEL)


# baseline (device time: 357743 ns/iter reference)
import jax
import jax.numpy as jnp
from jax import lax
from jax.experimental import pallas as pl
from jax.experimental.pallas import tpu as pltpu

N_DEV = 4
SQ = 2048
D_MODEL = 1024
HQ_LOCAL = 8
DH = 128
HD_LOCAL = HQ_LOCAL * DH
WINDOW = 128
SCALE = 0.08838834764831843
QBLK = 512
KBLK = QBLK + 2 * WINDOW


def _attn_body(x_ref, wq_ref, k_ref, v_ref, wo_ref, out_ref, ctx_ref):
    q = jnp.dot(x_ref[:], wq_ref[:], preferred_element_type=jnp.float32)
    for qb in range(SQ // QBLK):
        qs = qb * QBLK
        ks = min(max(qs - WINDOW, 0), SQ - KBLK)
        rows = lax.broadcasted_iota(jnp.int32, (QBLK, KBLK), 0) + qs
        cols = lax.broadcasted_iota(jnp.int32, (QBLK, KBLK), 1) + ks
        band = jnp.abs(rows - cols) <= WINDOW
        for h in range(HQ_LOCAL):
            qh = q[qs : qs + QBLK, h * DH : (h + 1) * DH]
            kh = k_ref[h, ks : ks + KBLK, :]
            s = lax.dot_general(
                qh, kh, (((1,), (1,)), ((), ())),
                preferred_element_type=jnp.float32,
            ) * SCALE
            s = jnp.where(band, s, -1e9)
            m = jnp.max(s, axis=1, keepdims=True)
            w = jnp.exp(s - m)
            w = w / jnp.sum(w, axis=1, keepdims=True)
            vh = v_ref[h, ks : ks + KBLK, :]
            ctx_ref[qs : qs + QBLK, h * DH : (h + 1) * DH] = jnp.dot(
                w, vh, preferred_element_type=jnp.float32
            )
    out_ref[:] = jnp.dot(ctx_ref[:], wo_ref[:], preferred_element_type=jnp.float32)


def _ar_body(p_ref, out_ref, comm_ref, send_sems, recv_sems):
    my_pos = lax.axis_index("i")
    left = (my_pos - 1) % N_DEV
    right = (my_pos + 1) % N_DEV

    barrier_sem = pltpu.get_barrier_semaphore()
    for nbr in [left, right]:
        pl.semaphore_signal(
            barrier_sem, inc=1,
            device_id=(nbr,), device_id_type=pl.DeviceIdType.MESH,
        )
    pl.semaphore_wait(barrier_sem, 2)

    out_ref[:] = p_ref[:]
    comm_ref[0] = p_ref[:]

    for h in range(N_DEV - 1):
        send_slot = h % 2
        recv_slot = (h + 1) % 2
        rdma = pltpu.make_async_remote_copy(
            src_ref=comm_ref.at[send_slot],
            dst_ref=comm_ref.at[recv_slot],
            send_sem=send_sems.at[send_slot],
            recv_sem=recv_sems.at[recv_slot],
            device_id=(right,),
            device_id_type=pl.DeviceIdType.MESH,
        )
        rdma.start()
        rdma.wait()
        out_ref[:] += comm_ref[recv_slot]


def kernel(x, Wq, K_ext, V_ext, Wo):
    my_pos = lax.axis_index("i")
    x2 = x.reshape(SQ, D_MODEL)
    wq_l = lax.dynamic_slice(Wq, (0, my_pos * HD_LOCAL), (D_MODEL, HD_LOCAL))
    wo_l = lax.dynamic_slice(Wo, (my_pos * HD_LOCAL, 0), (HD_LOCAL, D_MODEL))
    k = K_ext.reshape(SQ, HQ_LOCAL, DH).transpose(1, 0, 2)
    v = V_ext.reshape(SQ, HQ_LOCAL, DH).transpose(1, 0, 2)

    partial = pl.pallas_call(
        _attn_body,
        out_shape=jax.ShapeDtypeStruct((SQ, D_MODEL), jnp.float32),
        in_specs=[pl.BlockSpec(memory_space=pltpu.VMEM)] * 5,
        out_specs=pl.BlockSpec(memory_space=pltpu.VMEM),
        scratch_shapes=[pltpu.VMEM((SQ, HD_LOCAL), jnp.float32)],
        compiler_params=pltpu.CompilerParams(
            vmem_limit_bytes=100 * 1024 * 1024
        ),
    )(x2, wq_l, k, v, wo_l)

    out = pl.pallas_call(
        _ar_body,
        out_shape=jax.ShapeDtypeStruct((SQ, D_MODEL), jnp.float32),
        in_specs=[pl.BlockSpec(memory_space=pltpu.VMEM)],
        out_specs=pl.BlockSpec(memory_space=pltpu.VMEM),
        scratch_shapes=[
            pltpu.VMEM((2, SQ, D_MODEL), jnp.float32),
            pltpu.SemaphoreType.DMA((2,)),
            pltpu.SemaphoreType.DMA((2,)),
        ],
        compiler_params=pltpu.CompilerParams(
            collective_id=0, vmem_limit_bytes=100 * 1024 * 1024
        ),
    )(partial)
    return out.reshape(1, SQ, D_MODEL)


# device time: 157135 ns/iter; 2.2767x vs baseline; 2.2767x over previous
import jax
import jax.numpy as jnp
from jax import lax
from jax.experimental import pallas as pl
from jax.experimental.pallas import tpu as pltpu

N_DEV = 4
SQ = 2048
D_MODEL = 1024
HQ_LOCAL = 8
DH = 128
HD_LOCAL = HQ_LOCAL * DH
WINDOW = 128
SCALE = 0.08838834764831843
QBLK = 512
KBLK = QBLK + 2 * WINDOW


def _attn_body(x_ref, wq_ref, k_ref, v_ref, wo_ref, out_ref, ctx_ref):
    q = jnp.dot(x_ref[:], wq_ref[:], preferred_element_type=jnp.float32)
    for qb in range(SQ // QBLK):
        qs = qb * QBLK
        ks = min(max(qs - WINDOW, 0), SQ - KBLK)
        rows = lax.broadcasted_iota(jnp.int32, (QBLK, KBLK), 0) + qs
        cols = lax.broadcasted_iota(jnp.int32, (QBLK, KBLK), 1) + ks
        band = jnp.abs(rows - cols) <= WINDOW
        for h in range(HQ_LOCAL):
            qh = q[qs : qs + QBLK, h * DH : (h + 1) * DH]
            kh = k_ref[h, ks : ks + KBLK, :]
            s = lax.dot_general(
                qh, kh, (((1,), (1,)), ((), ())),
                preferred_element_type=jnp.float32,
            ) * SCALE
            s = jnp.where(band, s, -1e9)
            m = jnp.max(s, axis=1, keepdims=True)
            w = jnp.exp(s - m)
            w = w / jnp.sum(w, axis=1, keepdims=True)
            vh = v_ref[h, ks : ks + KBLK, :]
            ctx_ref[qs : qs + QBLK, h * DH : (h + 1) * DH] = jnp.dot(
                w, vh, preferred_element_type=jnp.float32
            )
    out_ref[:] = jnp.dot(ctx_ref[:], wo_ref[:], preferred_element_type=jnp.float32)


CHUNK = SQ // 2 // N_DEV


def _ar_body(p_ref, out_ref, cw_buf, ccw_buf, cw_s, cw_r, ccw_s, ccw_r):
    my = lax.axis_index("i")
    left = (my - 1) % N_DEV
    right = (my + 1) % N_DEV

    barrier_sem = pltpu.get_barrier_semaphore()
    for nbr in [left, right]:
        pl.semaphore_signal(
            barrier_sem, inc=1,
            device_id=(nbr,), device_id_type=pl.DeviceIdType.MESH,
        )
    pl.semaphore_wait(barrier_sem, 2)

    out_ref[:] = p_ref[:]

    def cw_rows(c):
        return pl.ds(c * CHUNK, CHUNK)

    def ccw_rows(c):
        return pl.ds(SQ // 2 + c * CHUNK, CHUNK)

    for s in range(N_DEV - 1):
        cw_send_c = (my - s) % N_DEV
        cw_recv_c = (my - s - 1) % N_DEV
        ccw_send_c = (my + s) % N_DEV
        ccw_recv_c = (my + s + 1) % N_DEV
        cw = pltpu.make_async_remote_copy(
            src_ref=out_ref.at[cw_rows(cw_send_c), :],
            dst_ref=cw_buf.at[s],
            send_sem=cw_s.at[s], recv_sem=cw_r.at[s],
            device_id=(right,), device_id_type=pl.DeviceIdType.MESH,
        )
        ccw = pltpu.make_async_remote_copy(
            src_ref=out_ref.at[ccw_rows(ccw_send_c), :],
            dst_ref=ccw_buf.at[s],
            send_sem=ccw_s.at[s], recv_sem=ccw_r.at[s],
            device_id=(left,), device_id_type=pl.DeviceIdType.MESH,
        )
        cw.start()
        ccw.start()
        cw.wait()
        ccw.wait()
        out_ref[cw_rows(cw_recv_c), :] += cw_buf[s]
        out_ref[ccw_rows(ccw_recv_c), :] += ccw_buf[s]

    for s in range(N_DEV - 1):
        r = s + N_DEV - 1
        cw_c = (my + 1 - s) % N_DEV
        ccw_c = (my - 1 + s) % N_DEV
        cw_in = (my - s) % N_DEV
        ccw_in = (my + s) % N_DEV
        cw_send = pltpu.make_async_remote_copy(
            src_ref=out_ref.at[cw_rows(cw_c), :],
            dst_ref=out_ref.at[cw_rows(cw_c), :],
            send_sem=cw_s.at[r], recv_sem=cw_r.at[r],
            device_id=(right,), device_id_type=pl.DeviceIdType.MESH,
        )
        ccw_send = pltpu.make_async_remote_copy(
            src_ref=out_ref.at[ccw_rows(ccw_c), :],
            dst_ref=out_ref.at[ccw_rows(ccw_c), :],
            send_sem=ccw_s.at[r], recv_sem=ccw_r.at[r],
            device_id=(left,), device_id_type=pl.DeviceIdType.MESH,
        )
        cw_send.start()
        ccw_send.start()
        cw_recv = pltpu.make_async_remote_copy(
            src_ref=out_ref.at[cw_rows(cw_in), :],
            dst_ref=out_ref.at[cw_rows(cw_in), :],
            send_sem=cw_s.at[r], recv_sem=cw_r.at[r],
            device_id=(left,), device_id_type=pl.DeviceIdType.MESH,
        )
        ccw_recv = pltpu.make_async_remote_copy(
            src_ref=out_ref.at[ccw_rows(ccw_in), :],
            dst_ref=out_ref.at[ccw_rows(ccw_in), :],
            send_sem=ccw_s.at[r], recv_sem=ccw_r.at[r],
            device_id=(right,), device_id_type=pl.DeviceIdType.MESH,
        )
        cw_send.wait_send()
        ccw_send.wait_send()
        cw_recv.wait_recv()
        ccw_recv.wait_recv()


def kernel(x, Wq, K_ext, V_ext, Wo):
    my_pos = lax.axis_index("i")
    x2 = x.reshape(SQ, D_MODEL)
    wq_l = lax.dynamic_slice(Wq, (0, my_pos * HD_LOCAL), (D_MODEL, HD_LOCAL))
    wo_l = lax.dynamic_slice(Wo, (my_pos * HD_LOCAL, 0), (HD_LOCAL, D_MODEL))
    k = K_ext.reshape(SQ, HQ_LOCAL, DH).transpose(1, 0, 2)
    v = V_ext.reshape(SQ, HQ_LOCAL, DH).transpose(1, 0, 2)

    partial = pl.pallas_call(
        _attn_body,
        out_shape=jax.ShapeDtypeStruct((SQ, D_MODEL), jnp.float32),
        in_specs=[pl.BlockSpec(memory_space=pltpu.VMEM)] * 5,
        out_specs=pl.BlockSpec(memory_space=pltpu.VMEM),
        scratch_shapes=[pltpu.VMEM((SQ, HD_LOCAL), jnp.float32)],
        compiler_params=pltpu.CompilerParams(
            vmem_limit_bytes=100 * 1024 * 1024
        ),
    )(x2, wq_l, k, v, wo_l)

    out = pl.pallas_call(
        _ar_body,
        out_shape=jax.ShapeDtypeStruct((SQ, D_MODEL), jnp.float32),
        in_specs=[pl.BlockSpec(memory_space=pltpu.VMEM)],
        out_specs=pl.BlockSpec(memory_space=pltpu.VMEM),
        scratch_shapes=[
            pltpu.VMEM((N_DEV - 1, CHUNK, D_MODEL), jnp.float32),
            pltpu.VMEM((N_DEV - 1, CHUNK, D_MODEL), jnp.float32),
            pltpu.SemaphoreType.DMA((2 * (N_DEV - 1),)),
            pltpu.SemaphoreType.DMA((2 * (N_DEV - 1),)),
            pltpu.SemaphoreType.DMA((2 * (N_DEV - 1),)),
            pltpu.SemaphoreType.DMA((2 * (N_DEV - 1),)),
        ],
        compiler_params=pltpu.CompilerParams(
            collective_id=0, vmem_limit_bytes=100 * 1024 * 1024
        ),
    )(partial)
    return out.reshape(1, SQ, D_MODEL)


# device time: 129724 ns/iter; 2.7577x vs baseline; 1.2113x over previous
import jax
import jax.numpy as jnp
from jax import lax
from jax.experimental import pallas as pl
from jax.experimental.pallas import tpu as pltpu

N_DEV = 4
SQ = 2048
D_MODEL = 1024
HQ_LOCAL = 8
DH = 128
HD_LOCAL = HQ_LOCAL * DH
WINDOW = 128
SCALE = 0.08838834764831843
CHUNK = SQ // 2 // N_DEV
KWIN = CHUNK + 2 * WINDOW


def _fused_body(
    x_ref, wq_ref, k_ref, v_ref, wo_ref, out_ref,
    cw_buf, ccw_buf, cw_s, cw_r, ccw_s, ccw_r,
):
    my = lax.axis_index("i")
    left = (my - 1) % N_DEV
    right = (my + 1) % N_DEV

    barrier_sem = pltpu.get_barrier_semaphore()
    for nbr in [left, right]:
        pl.semaphore_signal(
            barrier_sem, inc=1,
            device_id=(nbr,), device_id_type=pl.DeviceIdType.MESH,
        )
    pl.semaphore_wait(barrier_sem, 2)

    wq = wq_ref[:]
    wo = wo_ref[:]

    def compute_chunk(qs):
        qs = pl.multiple_of(qs, CHUNK)
        xq = x_ref[pl.ds(qs, CHUNK), :]
        q = jnp.dot(xq, wq, preferred_element_type=jnp.float32)
        ks = pl.multiple_of(jnp.clip(qs - WINDOW, 0, SQ - KWIN), WINDOW)
        rows = lax.broadcasted_iota(jnp.int32, (CHUNK, KWIN), 0) + qs
        cols = lax.broadcasted_iota(jnp.int32, (CHUNK, KWIN), 1) + ks
        band = jnp.abs(rows - cols) <= WINDOW
        ctx_parts = []
        for h in range(HQ_LOCAL):
            kh = k_ref[pl.ds(ks, KWIN), h * DH : (h + 1) * DH]
            s_ = lax.dot_general(
                q[:, h * DH : (h + 1) * DH], kh,
                (((1,), (1,)), ((), ())),
                preferred_element_type=jnp.float32,
            ) * SCALE
            s_ = jnp.where(band, s_, -1e9)
            m = jnp.max(s_, axis=1, keepdims=True)
            w = jnp.exp(s_ - m)
            w = w / jnp.sum(w, axis=1, keepdims=True)
            vh = v_ref[pl.ds(ks, KWIN), h * DH : (h + 1) * DH]
            ctx_parts.append(
                jnp.dot(w, vh, preferred_element_type=jnp.float32)
            )
        ctx = jnp.concatenate(ctx_parts, axis=1)
        return jnp.dot(ctx, wo, preferred_element_type=jnp.float32)

    def cw_rows(c):
        return pl.ds((c % N_DEV) * CHUNK, CHUNK)

    def ccw_rows(c):
        return pl.ds(SQ // 2 + (c % N_DEV) * CHUNK, CHUNK)

    out_ref[cw_rows(my), :] = compute_chunk((my % N_DEV) * CHUNK)
    out_ref[ccw_rows(my), :] = compute_chunk(SQ // 2 + (my % N_DEV) * CHUNK)

    for s in range(N_DEV - 1):
        cw_send_c = (my - s) % N_DEV
        cw_recv_c = (my - s - 1) % N_DEV
        ccw_send_c = (my + s) % N_DEV
        ccw_recv_c = (my + s + 1) % N_DEV
        cw = pltpu.make_async_remote_copy(
            src_ref=out_ref.at[cw_rows(cw_send_c), :],
            dst_ref=cw_buf.at[s],
            send_sem=cw_s.at[s], recv_sem=cw_r.at[s],
            device_id=(right,), device_id_type=pl.DeviceIdType.MESH,
        )
        ccw = pltpu.make_async_remote_copy(
            src_ref=out_ref.at[ccw_rows(ccw_send_c), :],
            dst_ref=ccw_buf.at[s],
            send_sem=ccw_s.at[s], recv_sem=ccw_r.at[s],
            device_id=(left,), device_id_type=pl.DeviceIdType.MESH,
        )
        cw.start()
        ccw.start()
        out_ref[cw_rows(cw_recv_c), :] = compute_chunk(cw_recv_c * CHUNK)
        out_ref[ccw_rows(ccw_recv_c), :] = compute_chunk(
            SQ // 2 + ccw_recv_c * CHUNK
        )
        cw.wait()
        ccw.wait()
        out_ref[cw_rows(cw_recv_c), :] += cw_buf[s]
        out_ref[ccw_rows(ccw_recv_c), :] += ccw_buf[s]

    for s in range(N_DEV - 1):
        r = s + N_DEV - 1
        cw_c = (my + 1 - s) % N_DEV
        ccw_c = (my - 1 + s) % N_DEV
        cw_in = (my - s) % N_DEV
        ccw_in = (my + s) % N_DEV
        cw_send = pltpu.make_async_remote_copy(
            src_ref=out_ref.at[cw_rows(cw_c), :],
            dst_ref=out_ref.at[cw_rows(cw_c), :],
            send_sem=cw_s.at[r], recv_sem=cw_r.at[r],
            device_id=(right,), device_id_type=pl.DeviceIdType.MESH,
        )
        ccw_send = pltpu.make_async_remote_copy(
            src_ref=out_ref.at[ccw_rows(ccw_c), :],
            dst_ref=out_ref.at[ccw_rows(ccw_c), :],
            send_sem=ccw_s.at[r], recv_sem=ccw_r.at[r],
            device_id=(left,), device_id_type=pl.DeviceIdType.MESH,
        )
        cw_send.start()
        ccw_send.start()
        cw_recv = pltpu.make_async_remote_copy(
            src_ref=out_ref.at[cw_rows(cw_in), :],
            dst_ref=out_ref.at[cw_rows(cw_in), :],
            send_sem=cw_s.at[r], recv_sem=cw_r.at[r],
            device_id=(left,), device_id_type=pl.DeviceIdType.MESH,
        )
        ccw_recv = pltpu.make_async_remote_copy(
            src_ref=out_ref.at[ccw_rows(ccw_in), :],
            dst_ref=out_ref.at[ccw_rows(ccw_in), :],
            send_sem=ccw_s.at[r], recv_sem=ccw_r.at[r],
            device_id=(right,), device_id_type=pl.DeviceIdType.MESH,
        )
        cw_send.wait_send()
        ccw_send.wait_send()
        cw_recv.wait_recv()
        ccw_recv.wait_recv()


def kernel(x, Wq, K_ext, V_ext, Wo):
    my_pos = lax.axis_index("i")
    x2 = x.reshape(SQ, D_MODEL)
    wq_l = lax.dynamic_slice(Wq, (0, my_pos * HD_LOCAL), (D_MODEL, HD_LOCAL))
    wo_l = lax.dynamic_slice(Wo, (my_pos * HD_LOCAL, 0), (HD_LOCAL, D_MODEL))
    k = K_ext.reshape(SQ, HD_LOCAL)
    v = V_ext.reshape(SQ, HD_LOCAL)

    out = pl.pallas_call(
        _fused_body,
        out_shape=jax.ShapeDtypeStruct((SQ, D_MODEL), jnp.float32),
        in_specs=[pl.BlockSpec(memory_space=pltpu.VMEM)] * 5,
        out_specs=pl.BlockSpec(memory_space=pltpu.VMEM),
        scratch_shapes=[
            pltpu.VMEM((N_DEV - 1, CHUNK, D_MODEL), jnp.float32),
            pltpu.VMEM((N_DEV - 1, CHUNK, D_MODEL), jnp.float32),
            pltpu.SemaphoreType.DMA((2 * (N_DEV - 1),)),
            pltpu.SemaphoreType.DMA((2 * (N_DEV - 1),)),
            pltpu.SemaphoreType.DMA((2 * (N_DEV - 1),)),
            pltpu.SemaphoreType.DMA((2 * (N_DEV - 1),)),
        ],
        compiler_params=pltpu.CompilerParams(
            collective_id=0, vmem_limit_bytes=62 * 1024 * 1024
        ),
    )(x2, wq_l, k, v, wo_l)
    return out.reshape(1, SQ, D_MODEL)


# device time: 124469 ns/iter; 2.8742x vs baseline; 1.0422x over previous
import jax
import jax.numpy as jnp
from jax import lax
from jax.experimental import pallas as pl
from jax.experimental.pallas import tpu as pltpu

N_DEV = 4
SQ = 2048
D_MODEL = 1024
HQ_LOCAL = 8
DH = 128
HD_LOCAL = HQ_LOCAL * DH
WINDOW = 128
SCALE = 0.08838834764831843
CHUNK = SQ // 2 // N_DEV
KWIN = CHUNK + 2 * WINDOW


def _fused_body(
    x_ref, wq_hbm, k_ref, v_ref, wo_hbm, out_ref,
    wq_vmem, wo_vmem, cw_buf, ccw_buf, w_sems, cw_s, cw_r, ccw_s, ccw_r,
):
    my = lax.axis_index("i")
    left = (my - 1) % N_DEV
    right = (my + 1) % N_DEV

    wq_dma = pltpu.make_async_copy(
        wq_hbm.at[:, pl.ds(my * HD_LOCAL, HD_LOCAL)], wq_vmem, w_sems.at[0]
    )
    wo_dma = pltpu.make_async_copy(
        wo_hbm.at[pl.ds(my * HD_LOCAL, HD_LOCAL), :], wo_vmem, w_sems.at[1]
    )
    wq_dma.start()
    wo_dma.start()

    barrier_sem = pltpu.get_barrier_semaphore()
    for nbr in [left, right]:
        pl.semaphore_signal(
            barrier_sem, inc=1,
            device_id=(nbr,), device_id_type=pl.DeviceIdType.MESH,
        )
    pl.semaphore_wait(barrier_sem, 2)

    wq_dma.wait()
    wo_dma.wait()
    wq = wq_vmem[:]
    wo = wo_vmem[:]

    def compute_chunk(qs):
        qs = pl.multiple_of(qs, CHUNK)
        xq = x_ref[pl.ds(qs, CHUNK), :]
        q = jnp.dot(xq, wq, preferred_element_type=jnp.float32)
        ks = pl.multiple_of(jnp.clip(qs - WINDOW, 0, SQ - KWIN), WINDOW)
        rows = lax.broadcasted_iota(jnp.int32, (CHUNK, KWIN), 0) + qs
        cols = lax.broadcasted_iota(jnp.int32, (CHUNK, KWIN), 1) + ks
        band = jnp.abs(rows - cols) <= WINDOW
        ctx_parts = []
        for h in range(HQ_LOCAL):
            kh = k_ref[pl.ds(ks, KWIN), h * DH : (h + 1) * DH]
            s_ = lax.dot_general(
                q[:, h * DH : (h + 1) * DH], kh,
                (((1,), (1,)), ((), ())),
                preferred_element_type=jnp.float32,
            ) * SCALE
            s_ = jnp.where(band, s_, -1e9)
            m = jnp.max(s_, axis=1, keepdims=True)
            w = jnp.exp(s_ - m)
            w = w / jnp.sum(w, axis=1, keepdims=True)
            vh = v_ref[pl.ds(ks, KWIN), h * DH : (h + 1) * DH]
            ctx_parts.append(
                jnp.dot(w, vh, preferred_element_type=jnp.float32)
            )
        ctx = jnp.concatenate(ctx_parts, axis=1)
        return jnp.dot(ctx, wo, preferred_element_type=jnp.float32)

    def cw_rows(c):
        return pl.ds((c % N_DEV) * CHUNK, CHUNK)

    def ccw_rows(c):
        return pl.ds(SQ // 2 + (c % N_DEV) * CHUNK, CHUNK)

    out_ref[cw_rows(my), :] = compute_chunk((my % N_DEV) * CHUNK)
    out_ref[ccw_rows(my), :] = compute_chunk(SQ // 2 + (my % N_DEV) * CHUNK)

    for s in range(N_DEV - 1):
        cw_send_c = (my - s) % N_DEV
        cw_recv_c = (my - s - 1) % N_DEV
        ccw_send_c = (my + s) % N_DEV
        ccw_recv_c = (my + s + 1) % N_DEV
        cw = pltpu.make_async_remote_copy(
            src_ref=out_ref.at[cw_rows(cw_send_c), :],
            dst_ref=cw_buf.at[s],
            send_sem=cw_s.at[s], recv_sem=cw_r.at[s],
            device_id=(right,), device_id_type=pl.DeviceIdType.MESH,
        )
        ccw = pltpu.make_async_remote_copy(
            src_ref=out_ref.at[ccw_rows(ccw_send_c), :],
            dst_ref=ccw_buf.at[s],
            send_sem=ccw_s.at[s], recv_sem=ccw_r.at[s],
            device_id=(left,), device_id_type=pl.DeviceIdType.MESH,
        )
        cw.start()
        ccw.start()
        out_ref[cw_rows(cw_recv_c), :] = compute_chunk(cw_recv_c * CHUNK)
        out_ref[ccw_rows(ccw_recv_c), :] = compute_chunk(
            SQ // 2 + ccw_recv_c * CHUNK
        )
        cw.wait()
        ccw.wait()
        out_ref[cw_rows(cw_recv_c), :] += cw_buf[s]
        out_ref[ccw_rows(ccw_recv_c), :] += ccw_buf[s]

    for s in range(N_DEV - 1):
        r = s + N_DEV - 1
        cw_c = (my + 1 - s) % N_DEV
        ccw_c = (my - 1 + s) % N_DEV
        cw_in = (my - s) % N_DEV
        ccw_in = (my + s) % N_DEV
        cw_send = pltpu.make_async_remote_copy(
            src_ref=out_ref.at[cw_rows(cw_c), :],
            dst_ref=out_ref.at[cw_rows(cw_c), :],
            send_sem=cw_s.at[r], recv_sem=cw_r.at[r],
            device_id=(right,), device_id_type=pl.DeviceIdType.MESH,
        )
        ccw_send = pltpu.make_async_remote_copy(
            src_ref=out_ref.at[ccw_rows(ccw_c), :],
            dst_ref=out_ref.at[ccw_rows(ccw_c), :],
            send_sem=ccw_s.at[r], recv_sem=ccw_r.at[r],
            device_id=(left,), device_id_type=pl.DeviceIdType.MESH,
        )
        cw_send.start()
        ccw_send.start()
        cw_recv = pltpu.make_async_remote_copy(
            src_ref=out_ref.at[cw_rows(cw_in), :],
            dst_ref=out_ref.at[cw_rows(cw_in), :],
            send_sem=cw_s.at[r], recv_sem=cw_r.at[r],
            device_id=(left,), device_id_type=pl.DeviceIdType.MESH,
        )
        ccw_recv = pltpu.make_async_remote_copy(
            src_ref=out_ref.at[ccw_rows(ccw_in), :],
            dst_ref=out_ref.at[ccw_rows(ccw_in), :],
            send_sem=ccw_s.at[r], recv_sem=ccw_r.at[r],
            device_id=(right,), device_id_type=pl.DeviceIdType.MESH,
        )
        cw_send.wait_send()
        ccw_send.wait_send()
        cw_recv.wait_recv()
        ccw_recv.wait_recv()


def kernel(x, Wq, K_ext, V_ext, Wo):
    x2 = x.reshape(SQ, D_MODEL)
    k = K_ext.reshape(SQ, HD_LOCAL)
    v = V_ext.reshape(SQ, HD_LOCAL)

    out = pl.pallas_call(
        _fused_body,
        out_shape=jax.ShapeDtypeStruct((SQ, D_MODEL), jnp.float32),
        in_specs=[
            pl.BlockSpec(memory_space=pltpu.VMEM),
            pl.BlockSpec(memory_space=pltpu.MemorySpace.HBM),
            pl.BlockSpec(memory_space=pltpu.VMEM),
            pl.BlockSpec(memory_space=pltpu.VMEM),
            pl.BlockSpec(memory_space=pltpu.MemorySpace.HBM),
        ],
        out_specs=pl.BlockSpec(memory_space=pltpu.VMEM),
        scratch_shapes=[
            pltpu.VMEM((D_MODEL, HD_LOCAL), jnp.float32),
            pltpu.VMEM((HD_LOCAL, D_MODEL), jnp.float32),
            pltpu.VMEM((N_DEV - 1, CHUNK, D_MODEL), jnp.float32),
            pltpu.VMEM((N_DEV - 1, CHUNK, D_MODEL), jnp.float32),
            pltpu.SemaphoreType.DMA((2,)),
            pltpu.SemaphoreType.DMA((2 * (N_DEV - 1),)),
            pltpu.SemaphoreType.DMA((2 * (N_DEV - 1),)),
            pltpu.SemaphoreType.DMA((2 * (N_DEV - 1),)),
            pltpu.SemaphoreType.DMA((2 * (N_DEV - 1),)),
        ],
        compiler_params=pltpu.CompilerParams(
            collective_id=0, vmem_limit_bytes=62 * 1024 * 1024
        ),
    )(x2, Wq, k, v, Wo)
    return out.reshape(1, SQ, D_MODEL)


# device time: 109151 ns/iter; 3.2775x vs baseline; 1.1403x over previous
import jax
import jax.numpy as jnp
from jax import lax
from jax.experimental import pallas as pl
from jax.experimental.pallas import tpu as pltpu

N_DEV = 4
SQ = 2048
D_MODEL = 1024
HQ_LOCAL = 8
DH = 128
HD_LOCAL = HQ_LOCAL * DH
WINDOW = 128
SCALE = 0.08838834764831843
CHUNK = SQ // 2 // N_DEV
KWIN = CHUNK + 2 * WINDOW


def _fused_body(
    x_ref, wq_hbm, k_hbm, v_hbm, wo_hbm, out_ref,
    wq_vmem, wo_vmem, k_ref, v_ref, cw_buf, ccw_buf,
    w_sems, kv_sems, cw_s, cw_r, ccw_s, ccw_r,
):
    my = lax.axis_index("i")
    left = (my - 1) % N_DEV
    right = (my + 1) % N_DEV

    wq_dma = pltpu.make_async_copy(
        wq_hbm.at[:, pl.ds(my * HD_LOCAL, HD_LOCAL)], wq_vmem, w_sems.at[0]
    )
    wo_dma = pltpu.make_async_copy(
        wo_hbm.at[pl.ds(my * HD_LOCAL, HD_LOCAL), :], wo_vmem, w_sems.at[1]
    )
    wq_dma.start()
    wo_dma.start()
    kv_dmas = []
    for h in range(HQ_LOCAL):
        kd = pltpu.make_async_copy(
            k_hbm.at[:, h, :], k_ref.at[:, h * DH : (h + 1) * DH],
            kv_sems.at[h],
        )
        vd = pltpu.make_async_copy(
            v_hbm.at[:, h, :], v_ref.at[:, h * DH : (h + 1) * DH],
            kv_sems.at[HQ_LOCAL + h],
        )
        kd.start()
        vd.start()
        kv_dmas += [kd, vd]

    barrier_sem = pltpu.get_barrier_semaphore()
    for nbr in [left, right]:
        pl.semaphore_signal(
            barrier_sem, inc=1,
            device_id=(nbr,), device_id_type=pl.DeviceIdType.MESH,
        )
    pl.semaphore_wait(barrier_sem, 2)

    wq_dma.wait()
    wo_dma.wait()
    for d in kv_dmas:
        d.wait()
    wq = wq_vmem[:]
    wo = wo_vmem[:]

    def compute_chunk(qs):
        qs = pl.multiple_of(qs, CHUNK)
        xq = x_ref[pl.ds(qs, CHUNK), :]
        q = jnp.dot(xq, wq, preferred_element_type=jnp.float32)
        ks = pl.multiple_of(jnp.clip(qs - WINDOW, 0, SQ - KWIN), WINDOW)
        rows = lax.broadcasted_iota(jnp.int32, (CHUNK, KWIN), 0) + qs
        cols = lax.broadcasted_iota(jnp.int32, (CHUNK, KWIN), 1) + ks
        band = jnp.abs(rows - cols) <= WINDOW
        ctx_parts = []
        for h in range(HQ_LOCAL):
            kh = k_ref[pl.ds(ks, KWIN), h * DH : (h + 1) * DH]
            s_ = lax.dot_general(
                q[:, h * DH : (h + 1) * DH], kh,
                (((1,), (1,)), ((), ())),
                preferred_element_type=jnp.float32,
            ) * SCALE
            s_ = jnp.where(band, s_, -1e9)
            m = jnp.max(s_, axis=1, keepdims=True)
            w = jnp.exp(s_ - m)
            w = w / jnp.sum(w, axis=1, keepdims=True)
            vh = v_ref[pl.ds(ks, KWIN), h * DH : (h + 1) * DH]
            ctx_parts.append(
                jnp.dot(w, vh, preferred_element_type=jnp.float32)
            )
        ctx = jnp.concatenate(ctx_parts, axis=1)
        return jnp.dot(ctx, wo, preferred_element_type=jnp.float32)

    def cw_rows(c):
        return pl.ds((c % N_DEV) * CHUNK, CHUNK)

    def ccw_rows(c):
        return pl.ds(SQ // 2 + (c % N_DEV) * CHUNK, CHUNK)

    out_ref[cw_rows(my), :] = compute_chunk((my % N_DEV) * CHUNK)
    out_ref[ccw_rows(my), :] = compute_chunk(SQ // 2 + (my % N_DEV) * CHUNK)

    for s in range(N_DEV - 1):
        cw_send_c = (my - s) % N_DEV
        cw_recv_c = (my - s - 1) % N_DEV
        ccw_send_c = (my + s) % N_DEV
        ccw_recv_c = (my + s + 1) % N_DEV
        cw = pltpu.make_async_remote_copy(
            src_ref=out_ref.at[cw_rows(cw_send_c), :],
            dst_ref=cw_buf.at[s],
            send_sem=cw_s.at[s], recv_sem=cw_r.at[s],
            device_id=(right,), device_id_type=pl.DeviceIdType.MESH,
        )
        ccw = pltpu.make_async_remote_copy(
            src_ref=out_ref.at[ccw_rows(ccw_send_c), :],
            dst_ref=ccw_buf.at[s],
            send_sem=ccw_s.at[s], recv_sem=ccw_r.at[s],
            device_id=(left,), device_id_type=pl.DeviceIdType.MESH,
        )
        cw.start()
        ccw.start()
        out_ref[cw_rows(cw_recv_c), :] = compute_chunk(cw_recv_c * CHUNK)
        out_ref[ccw_rows(ccw_recv_c), :] = compute_chunk(
            SQ // 2 + ccw_recv_c * CHUNK
        )
        cw.wait()
        ccw.wait()
        out_ref[cw_rows(cw_recv_c), :] += cw_buf[s]
        out_ref[ccw_rows(ccw_recv_c), :] += ccw_buf[s]

    for s in range(N_DEV - 1):
        r = s + N_DEV - 1
        cw_c = (my + 1 - s) % N_DEV
        ccw_c = (my - 1 + s) % N_DEV
        cw_in = (my - s) % N_DEV
        ccw_in = (my + s) % N_DEV
        cw_send = pltpu.make_async_remote_copy(
            src_ref=out_ref.at[cw_rows(cw_c), :],
            dst_ref=out_ref.at[cw_rows(cw_c), :],
            send_sem=cw_s.at[r], recv_sem=cw_r.at[r],
            device_id=(right,), device_id_type=pl.DeviceIdType.MESH,
        )
        ccw_send = pltpu.make_async_remote_copy(
            src_ref=out_ref.at[ccw_rows(ccw_c), :],
            dst_ref=out_ref.at[ccw_rows(ccw_c), :],
            send_sem=ccw_s.at[r], recv_sem=ccw_r.at[r],
            device_id=(left,), device_id_type=pl.DeviceIdType.MESH,
        )
        cw_send.start()
        ccw_send.start()
        cw_recv = pltpu.make_async_remote_copy(
            src_ref=out_ref.at[cw_rows(cw_in), :],
            dst_ref=out_ref.at[cw_rows(cw_in), :],
            send_sem=cw_s.at[r], recv_sem=cw_r.at[r],
            device_id=(left,), device_id_type=pl.DeviceIdType.MESH,
        )
        ccw_recv = pltpu.make_async_remote_copy(
            src_ref=out_ref.at[ccw_rows(ccw_in), :],
            dst_ref=out_ref.at[ccw_rows(ccw_in), :],
            send_sem=ccw_s.at[r], recv_sem=ccw_r.at[r],
            device_id=(right,), device_id_type=pl.DeviceIdType.MESH,
        )
        cw_send.wait_send()
        ccw_send.wait_send()
        cw_recv.wait_recv()
        ccw_recv.wait_recv()


def kernel(x, Wq, K_ext, V_ext, Wo):
    x2 = x.reshape(SQ, D_MODEL)
    k = K_ext.reshape(SQ, HQ_LOCAL, DH)
    v = V_ext.reshape(SQ, HQ_LOCAL, DH)

    out = pl.pallas_call(
        _fused_body,
        out_shape=jax.ShapeDtypeStruct((SQ, D_MODEL), jnp.float32),
        in_specs=[
            pl.BlockSpec(memory_space=pltpu.VMEM),
            pl.BlockSpec(memory_space=pltpu.MemorySpace.HBM),
            pl.BlockSpec(memory_space=pltpu.MemorySpace.HBM),
            pl.BlockSpec(memory_space=pltpu.MemorySpace.HBM),
            pl.BlockSpec(memory_space=pltpu.MemorySpace.HBM),
        ],
        out_specs=pl.BlockSpec(memory_space=pltpu.VMEM),
        scratch_shapes=[
            pltpu.VMEM((D_MODEL, HD_LOCAL), jnp.float32),
            pltpu.VMEM((HD_LOCAL, D_MODEL), jnp.float32),
            pltpu.VMEM((SQ, HD_LOCAL), jnp.float32),
            pltpu.VMEM((SQ, HD_LOCAL), jnp.float32),
            pltpu.VMEM((N_DEV - 1, CHUNK, D_MODEL), jnp.float32),
            pltpu.VMEM((N_DEV - 1, CHUNK, D_MODEL), jnp.float32),
            pltpu.SemaphoreType.DMA((2,)),
            pltpu.SemaphoreType.DMA((2 * HQ_LOCAL,)),
            pltpu.SemaphoreType.DMA((2 * (N_DEV - 1),)),
            pltpu.SemaphoreType.DMA((2 * (N_DEV - 1),)),
            pltpu.SemaphoreType.DMA((2 * (N_DEV - 1),)),
            pltpu.SemaphoreType.DMA((2 * (N_DEV - 1),)),
        ],
        compiler_params=pltpu.CompilerParams(
            collective_id=0, vmem_limit_bytes=62 * 1024 * 1024
        ),
    )(x2, Wq, k, v, Wo)
    return out.reshape(1, SQ, D_MODEL)


# device time: 105001 ns/iter; 3.4070x vs baseline; 1.0395x over previous
import jax
import jax.numpy as jnp
from jax import lax
from jax.experimental import pallas as pl
from jax.experimental.pallas import tpu as pltpu

N_DEV = 4
SQ = 2048
D_MODEL = 1024
HQ_LOCAL = 8
DH = 128
HD_LOCAL = HQ_LOCAL * DH
WINDOW = 128
SCALE = 0.08838834764831843
CHUNK = SQ // 2 // N_DEV
KWIN = CHUNK + 2 * WINDOW


SUB = CHUNK // 2


def _fused_body(
    x_hbm, wq_hbm, k_hbm, v_hbm, wo_hbm, out_ref,
    x_ref, wq_vmem, wo_vmem, k_ref, v_ref, cw_buf, ccw_buf,
    w_sems, kv_sems, cw_s, cw_r, ccw_s, ccw_r,
):
    my = lax.axis_index("i")
    left = (my - 1) % N_DEV
    right = (my + 1) % N_DEV

    x_dma = pltpu.make_async_copy(x_hbm, x_ref, w_sems.at[2])
    x_dma.start()
    wq_dma = pltpu.make_async_copy(
        wq_hbm.at[:, pl.ds(my * HD_LOCAL, HD_LOCAL)], wq_vmem, w_sems.at[0]
    )
    wo_dma = pltpu.make_async_copy(
        wo_hbm.at[pl.ds(my * HD_LOCAL, HD_LOCAL), :], wo_vmem, w_sems.at[1]
    )
    wq_dma.start()
    wo_dma.start()
    kv_dmas = []
    for h in range(HQ_LOCAL):
        kd = pltpu.make_async_copy(
            k_hbm.at[:, h, :], k_ref.at[:, h * DH : (h + 1) * DH],
            kv_sems.at[h],
        )
        vd = pltpu.make_async_copy(
            v_hbm.at[:, h, :], v_ref.at[:, h * DH : (h + 1) * DH],
            kv_sems.at[HQ_LOCAL + h],
        )
        kd.start()
        vd.start()
        kv_dmas += [kd, vd]

    barrier_sem = pltpu.get_barrier_semaphore()
    for nbr in [left, right]:
        pl.semaphore_signal(
            barrier_sem, inc=1,
            device_id=(nbr,), device_id_type=pl.DeviceIdType.MESH,
        )
    pl.semaphore_wait(barrier_sem, 2)

    x_dma.wait()
    wq_dma.wait()
    wo_dma.wait()
    for d in kv_dmas:
        d.wait()
    wq = wq_vmem[:]
    wo = wo_vmem[:]

    def compute_chunk(qs):
        qs = pl.multiple_of(qs, CHUNK)
        xq = x_ref[pl.ds(qs, CHUNK), :]
        q = jnp.dot(xq, wq, preferred_element_type=jnp.float32)
        ks = pl.multiple_of(jnp.clip(qs - WINDOW, 0, SQ - KWIN), WINDOW)
        rows = lax.broadcasted_iota(jnp.int32, (CHUNK, KWIN), 0) + qs
        cols = lax.broadcasted_iota(jnp.int32, (CHUNK, KWIN), 1) + ks
        band = jnp.abs(rows - cols) <= WINDOW
        ctx_parts = []
        for h in range(HQ_LOCAL):
            kh = k_ref[pl.ds(ks, KWIN), h * DH : (h + 1) * DH]
            s_ = lax.dot_general(
                q[:, h * DH : (h + 1) * DH], kh,
                (((1,), (1,)), ((), ())),
                preferred_element_type=jnp.float32,
            ) * SCALE
            s_ = jnp.where(band, s_, -1e9)
            m = jnp.max(s_, axis=1, keepdims=True)
            w = jnp.exp(s_ - m)
            w = w / jnp.sum(w, axis=1, keepdims=True)
            vh = v_ref[pl.ds(ks, KWIN), h * DH : (h + 1) * DH]
            ctx_parts.append(
                jnp.dot(w, vh, preferred_element_type=jnp.float32)
            )
        ctx = jnp.concatenate(ctx_parts, axis=1)
        return jnp.dot(ctx, wo, preferred_element_type=jnp.float32)

    def cw_rows(c):
        return pl.ds((c % N_DEV) * CHUNK, CHUNK)

    def ccw_rows(c):
        return pl.ds(SQ // 2 + (c % N_DEV) * CHUNK, CHUNK)

    out_ref[cw_rows(my), :] = compute_chunk((my % N_DEV) * CHUNK)
    out_ref[ccw_rows(my), :] = compute_chunk(SQ // 2 + (my % N_DEV) * CHUNK)

    for s in range(N_DEV - 1):
        cw_send_c = (my - s) % N_DEV
        cw_recv_c = (my - s - 1) % N_DEV
        ccw_send_c = (my + s) % N_DEV
        ccw_recv_c = (my + s + 1) % N_DEV
        cw = pltpu.make_async_remote_copy(
            src_ref=out_ref.at[cw_rows(cw_send_c), :],
            dst_ref=cw_buf.at[s],
            send_sem=cw_s.at[s], recv_sem=cw_r.at[s],
            device_id=(right,), device_id_type=pl.DeviceIdType.MESH,
        )
        ccw = pltpu.make_async_remote_copy(
            src_ref=out_ref.at[ccw_rows(ccw_send_c), :],
            dst_ref=ccw_buf.at[s],
            send_sem=ccw_s.at[s], recv_sem=ccw_r.at[s],
            device_id=(left,), device_id_type=pl.DeviceIdType.MESH,
        )
        cw.start()
        ccw.start()
        out_ref[cw_rows(cw_recv_c), :] = compute_chunk(cw_recv_c * CHUNK)
        out_ref[ccw_rows(ccw_recv_c), :] = compute_chunk(
            SQ // 2 + ccw_recv_c * CHUNK
        )
        cw.wait()
        ccw.wait()
        out_ref[cw_rows(cw_recv_c), :] += cw_buf[s]
        out_ref[ccw_rows(ccw_recv_c), :] += ccw_buf[s]

    def cw_sub(c, sub):
        return pl.ds((c % N_DEV) * CHUNK + sub * SUB, SUB)

    def ccw_sub(c, sub):
        return pl.ds(SQ // 2 + (c % N_DEV) * CHUNK + sub * SUB, SUB)

    def ag_send(t):
        s, sub = divmod(t, 2)
        r = N_DEV - 1 + t
        cw_c = (my + 1 - s) % N_DEV
        ccw_c = (my - 1 + s) % N_DEV
        cw = pltpu.make_async_remote_copy(
            src_ref=out_ref.at[cw_sub(cw_c, sub), :],
            dst_ref=out_ref.at[cw_sub(cw_c, sub), :],
            send_sem=cw_s.at[r], recv_sem=cw_r.at[r],
            device_id=(right,), device_id_type=pl.DeviceIdType.MESH,
        )
        ccw = pltpu.make_async_remote_copy(
            src_ref=out_ref.at[ccw_sub(ccw_c, sub), :],
            dst_ref=out_ref.at[ccw_sub(ccw_c, sub), :],
            send_sem=ccw_s.at[r], recv_sem=ccw_r.at[r],
            device_id=(left,), device_id_type=pl.DeviceIdType.MESH,
        )
        cw.start()
        ccw.start()
        return cw, ccw

    def ag_recv(t):
        s, sub = divmod(t, 2)
        r = N_DEV - 1 + t
        cw_in = (my - s) % N_DEV
        ccw_in = (my + s) % N_DEV
        cw = pltpu.make_async_remote_copy(
            src_ref=out_ref.at[cw_sub(cw_in, sub), :],
            dst_ref=out_ref.at[cw_sub(cw_in, sub), :],
            send_sem=cw_s.at[r], recv_sem=cw_r.at[r],
            device_id=(left,), device_id_type=pl.DeviceIdType.MESH,
        )
        ccw = pltpu.make_async_remote_copy(
            src_ref=out_ref.at[ccw_sub(ccw_in, sub), :],
            dst_ref=out_ref.at[ccw_sub(ccw_in, sub), :],
            send_sem=ccw_s.at[r], recv_sem=ccw_r.at[r],
            device_id=(right,), device_id_type=pl.DeviceIdType.MESH,
        )
        return cw, ccw

    n_msg = 2 * (N_DEV - 1)
    sends = [ag_send(0), ag_send(1)]
    for t in range(n_msg):
        rcw, rccw = ag_recv(t)
        rcw.wait_recv()
        rccw.wait_recv()
        if t + 2 < n_msg:
            sends.append(ag_send(t + 2))
    for scw, sccw in sends:
        scw.wait_send()
        sccw.wait_send()


def kernel(x, Wq, K_ext, V_ext, Wo):
    x2 = x.reshape(SQ, D_MODEL)
    k = K_ext.reshape(SQ, HQ_LOCAL, DH)
    v = V_ext.reshape(SQ, HQ_LOCAL, DH)

    out = pl.pallas_call(
        _fused_body,
        out_shape=jax.ShapeDtypeStruct((SQ, D_MODEL), jnp.float32),
        in_specs=[
            pl.BlockSpec(memory_space=pltpu.MemorySpace.HBM),
            pl.BlockSpec(memory_space=pltpu.MemorySpace.HBM),
            pl.BlockSpec(memory_space=pltpu.MemorySpace.HBM),
            pl.BlockSpec(memory_space=pltpu.MemorySpace.HBM),
            pl.BlockSpec(memory_space=pltpu.MemorySpace.HBM),
        ],
        out_specs=pl.BlockSpec(memory_space=pltpu.VMEM),
        scratch_shapes=[
            pltpu.VMEM((SQ, D_MODEL), jnp.float32),
            pltpu.VMEM((D_MODEL, HD_LOCAL), jnp.float32),
            pltpu.VMEM((HD_LOCAL, D_MODEL), jnp.float32),
            pltpu.VMEM((SQ, HD_LOCAL), jnp.float32),
            pltpu.VMEM((SQ, HD_LOCAL), jnp.float32),
            pltpu.VMEM((N_DEV - 1, CHUNK, D_MODEL), jnp.float32),
            pltpu.VMEM((N_DEV - 1, CHUNK, D_MODEL), jnp.float32),
            pltpu.SemaphoreType.DMA((3,)),
            pltpu.SemaphoreType.DMA((2 * HQ_LOCAL,)),
            pltpu.SemaphoreType.DMA((3 * (N_DEV - 1),)),
            pltpu.SemaphoreType.DMA((3 * (N_DEV - 1),)),
            pltpu.SemaphoreType.DMA((3 * (N_DEV - 1),)),
            pltpu.SemaphoreType.DMA((3 * (N_DEV - 1),)),
        ],
        compiler_params=pltpu.CompilerParams(
            collective_id=0, vmem_limit_bytes=62 * 1024 * 1024
        ),
    )(x2, Wq, k, v, Wo)
    return out.reshape(1, SQ, D_MODEL)
